# initial kernel scaffold (unmeasured)
import jax
import jax.numpy as jnp
from jax import lax
from jax.experimental import pallas as pl
from jax.experimental.pallas import tpu as pltpu


def kernel(
    x,
):
    def body(*refs):
        pass

    out_shape = jax.ShapeDtypeStruct(..., jnp.float32)
    return pl.pallas_call(body, out_shape=out_shape)(...)


# baseline (device time: 30389 ns/iter reference)
import jax
import jax.numpy as jnp
from jax import lax
from jax.experimental import pallas as pl
from jax.experimental.pallas import tpu as pltpu

M_PER = 2048
N = 512


def kernel(x):
    m_per, n = x.shape

    def body(x_ref, out_ref, send_sem, recv_sem):
        my_x = lax.axis_index("x")
        my_y = lax.axis_index("y")
        peer = (1 - my_x, my_y)

        barrier_sem = pltpu.get_barrier_semaphore()
        pl.semaphore_signal(
            barrier_sem, inc=1, device_id=peer,
            device_id_type=pl.DeviceIdType.MESH,
        )
        pl.semaphore_wait(barrier_sem, 1)

        my_rows = pl.ds(my_x * m_per, m_per)
        out_ref[my_rows, :] = x_ref[:, :].astype(jnp.bfloat16)

        rdma = pltpu.make_async_remote_copy(
            src_ref=out_ref.at[my_rows, :],
            dst_ref=out_ref.at[my_rows, :],
            send_sem=send_sem,
            recv_sem=recv_sem,
            device_id=peer,
            device_id_type=pl.DeviceIdType.MESH,
        )
        rdma.start()
        rdma.wait()

    return pl.pallas_call(
        body,
        out_shape=jax.ShapeDtypeStruct((2 * m_per, n), jnp.bfloat16),
        in_specs=[pl.BlockSpec(memory_space=pltpu.VMEM)],
        out_specs=pl.BlockSpec(memory_space=pltpu.VMEM),
        scratch_shapes=[
            pltpu.SemaphoreType.DMA,
            pltpu.SemaphoreType.DMA,
        ],
        compiler_params=pltpu.CompilerParams(collective_id=0),
    )(x)


# device time: 23157 ns/iter; 1.3123x vs baseline; 1.3123x over previous
import jax
import jax.numpy as jnp
from jax import lax
from jax.experimental import pallas as pl
from jax.experimental.pallas import tpu as pltpu

M_PER = 2048
HALF = M_PER // 2
N = 512
K = 8
BLK = HALF // K


def kernel(x):
    m_per, n = x.shape

    def body(x_ref, out_ref, xs_sems, xr_sems, ys_sems, yr_sems):
        my_x = lax.axis_index("x")
        my_y = lax.axis_index("y")
        x_peer = (1 - my_x, my_y)
        y_peer = (my_x, 1 - my_y)

        barrier_sem = pltpu.get_barrier_semaphore()
        for nbr in (x_peer, y_peer):
            pl.semaphore_signal(
                barrier_sem, inc=1, device_id=nbr,
                device_id_type=pl.DeviceIdType.MESH,
            )
        pl.semaphore_wait(barrier_sem, 2)

        my_base = my_x * M_PER
        send_off = my_base + my_y * HALF
        keep_off = my_base + (1 - my_y) * HALF
        out_ref[pl.ds(send_off, HALF), :] = (
            x_ref[pl.ds(my_y * HALF, HALF), :].astype(jnp.bfloat16)
        )

        p1 = []
        for b in range(K):
            rows = pl.ds(send_off + b * BLK, BLK)
            r = pltpu.make_async_remote_copy(
                src_ref=out_ref.at[rows, :],
                dst_ref=out_ref.at[rows, :],
                send_sem=xs_sems.at[b],
                recv_sem=xr_sems.at[b],
                device_id=x_peer,
                device_id_type=pl.DeviceIdType.MESH,
            )
            r.start()
            p1.append(r)

        out_ref[pl.ds(keep_off, HALF), :] = (
            x_ref[pl.ds((1 - my_y) * HALF, HALF), :].astype(jnp.bfloat16)
        )

        recv1_off = (1 - my_x) * M_PER + my_y * HALF
        p2 = []
        for b in range(K):
            p1[b].wait_recv()
            rows = pl.ds(recv1_off + b * BLK, BLK)
            f = pltpu.make_async_remote_copy(
                src_ref=out_ref.at[rows, :],
                dst_ref=out_ref.at[rows, :],
                send_sem=ys_sems.at[b],
                recv_sem=yr_sems.at[b],
                device_id=y_peer,
                device_id_type=pl.DeviceIdType.MESH,
            )
            f.start()
            p2.append(f)

        for b in range(K):
            p1[b].wait_send()
            p2[b].wait_send()
            p2[b].wait_recv()

    return pl.pallas_call(
        body,
        out_shape=jax.ShapeDtypeStruct((2 * m_per, n), jnp.bfloat16),
        in_specs=[pl.BlockSpec(memory_space=pltpu.VMEM)],
        out_specs=pl.BlockSpec(memory_space=pltpu.VMEM),
        scratch_shapes=[
            pltpu.SemaphoreType.DMA((K,)),
            pltpu.SemaphoreType.DMA((K,)),
            pltpu.SemaphoreType.DMA((K,)),
            pltpu.SemaphoreType.DMA((K,)),
        ],
        compiler_params=pltpu.CompilerParams(collective_id=0),
    )(x)


# device time: 22650 ns/iter; 1.3417x vs baseline; 1.0224x over previous
import jax
import jax.numpy as jnp
from jax import lax
from jax.experimental import pallas as pl
from jax.experimental.pallas import tpu as pltpu

M_PER = 2048
HALF = M_PER // 2
N = 512
K = 16
BLK = HALF // K


def kernel(x):
    m_per, n = x.shape

    def body(x_ref, out_ref, xs_sems, xr_sems, ys_sems, yr_sems):
        my_x = lax.axis_index("x")
        my_y = lax.axis_index("y")
        x_peer = (1 - my_x, my_y)
        y_peer = (my_x, 1 - my_y)

        barrier_sem = pltpu.get_barrier_semaphore()
        for nbr in (x_peer, y_peer):
            pl.semaphore_signal(
                barrier_sem, inc=1, device_id=nbr,
                device_id_type=pl.DeviceIdType.MESH,
            )
        pl.semaphore_wait(barrier_sem, 2)

        my_base = my_x * M_PER
        send_off = my_base + my_y * HALF
        keep_off = my_base + (1 - my_y) * HALF

        p1 = []
        for b in range(K):
            rows = pl.ds(send_off + b * BLK, BLK)
            out_ref[rows, :] = (
                x_ref[pl.ds(my_y * HALF + b * BLK, BLK), :].astype(jnp.bfloat16)
            )
            r = pltpu.make_async_remote_copy(
                src_ref=out_ref.at[rows, :],
                dst_ref=out_ref.at[rows, :],
                send_sem=xs_sems.at[b],
                recv_sem=xr_sems.at[b],
                device_id=x_peer,
                device_id_type=pl.DeviceIdType.MESH,
            )
            r.start()
            p1.append(r)

        out_ref[pl.ds(keep_off, HALF), :] = (
            x_ref[pl.ds((1 - my_y) * HALF, HALF), :].astype(jnp.bfloat16)
        )

        recv1_off = (1 - my_x) * M_PER + my_y * HALF
        p2 = []
        for b in range(K):
            p1[b].wait_recv()
            rows = pl.ds(recv1_off + b * BLK, BLK)
            f = pltpu.make_async_remote_copy(
                src_ref=out_ref.at[rows, :],
                dst_ref=out_ref.at[rows, :],
                send_sem=ys_sems.at[b],
                recv_sem=yr_sems.at[b],
                device_id=y_peer,
                device_id_type=pl.DeviceIdType.MESH,
            )
            f.start()
            p2.append(f)

        for b in range(K):
            p1[b].wait_send()
            p2[b].wait_send()
            p2[b].wait_recv()

    return pl.pallas_call(
        body,
        out_shape=jax.ShapeDtypeStruct((2 * m_per, n), jnp.bfloat16),
        in_specs=[pl.BlockSpec(memory_space=pltpu.VMEM)],
        out_specs=pl.BlockSpec(memory_space=pltpu.VMEM),
        scratch_shapes=[
            pltpu.SemaphoreType.DMA((K,)),
            pltpu.SemaphoreType.DMA((K,)),
            pltpu.SemaphoreType.DMA((K,)),
            pltpu.SemaphoreType.DMA((K,)),
        ],
        compiler_params=pltpu.CompilerParams(collective_id=0),
    )(x)


# device time: 20516 ns/iter; 1.4812x vs baseline; 1.1040x over previous
import jax
import jax.numpy as jnp
from jax import lax
from jax.experimental import pallas as pl
from jax.experimental.pallas import tpu as pltpu

M_PER = 2048
N = 512
RX = 1152
RY = 896
BLK = 64
KX = RX // BLK
KF = RY // BLK

_OFF0 = [i * BLK for i in range(KX)]
_OFF1 = [RX + i * BLK for i in range(KF)] + [
    RY + (i - KF) * BLK for i in range(KF, KX)
]


def kernel(x):
    m_per, n = x.shape

    def body(x_hbm, out_ref, xf32_ref, mine_ref, recv_ref, xs_sems,
             xr_sems, ys_sems, yr_sems, ld_sems, cp_sems, cp2_sems):
        my_x = lax.axis_index("x")
        my_y = lax.axis_index("y")
        x_peer = (1 - my_x, my_y)
        y_peer = (my_x, 1 - my_y)

        def off(i):
            return _OFF0[i] + my_y * (_OFF1[i] - _OFF0[i])

        loads = []
        for i in range(KX):
            rows = pl.ds(off(i), BLK)
            ld = pltpu.make_async_copy(
                x_hbm.at[rows, :], xf32_ref.at[rows, :], ld_sems.at[i]
            )
            ld.start()
            loads.append(ld)
        rem_rows = pl.ds((1 - my_y) * RX, RY)
        ld_rem = pltpu.make_async_copy(
            x_hbm.at[rem_rows, :], xf32_ref.at[rem_rows, :], ld_sems.at[KX]
        )
        ld_rem.start()

        barrier_sem = pltpu.get_barrier_semaphore()
        for nbr in (x_peer, y_peer):
            pl.semaphore_signal(
                barrier_sem, inc=1, device_id=nbr,
                device_id_type=pl.DeviceIdType.MESH,
            )
        pl.semaphore_wait(barrier_sem, 2)

        p1 = []
        for i in range(KX):
            rows = pl.ds(off(i), BLK)
            loads[i].wait()
            mine_ref[rows, :] = xf32_ref[rows, :].astype(jnp.bfloat16)
            r = pltpu.make_async_remote_copy(
                src_ref=mine_ref.at[rows, :],
                dst_ref=recv_ref.at[rows, :],
                send_sem=xs_sems.at[i],
                recv_sem=xr_sems.at[i],
                device_id=x_peer,
                device_id_type=pl.DeviceIdType.MESH,
            )
            r.start()
            p1.append(r)

        ld_rem.wait()
        mine_ref[rem_rows, :] = xf32_ref[rem_rows, :].astype(jnp.bfloat16)
        cp_mine = pltpu.make_async_copy(
            mine_ref, out_ref.at[pl.ds(my_x * M_PER, M_PER), :], cp_sems.at[0]
        )
        cp_mine.start()

        base = (1 - my_x) * M_PER
        p2 = []
        cps = []
        for i in range(KX):
            p1[i].wait_recv()
            rows = pl.ds(off(i), BLK)
            orows = pl.ds(base + off(i), BLK)
            if i < KF:
                f = pltpu.make_async_remote_copy(
                    src_ref=recv_ref.at[rows, :],
                    dst_ref=out_ref.at[orows, :],
                    send_sem=ys_sems.at[i],
                    recv_sem=yr_sems.at[i],
                    device_id=y_peer,
                    device_id_type=pl.DeviceIdType.MESH,
                )
                f.start()
                p2.append(f)
            cp = pltpu.make_async_copy(
                recv_ref.at[rows, :], out_ref.at[orows, :], cp2_sems.at[i]
            )
            cp.start()
            cps.append(cp)

        for i in range(KX):
            p1[i].wait_send()
            cps[i].wait()
        for f in p2:
            f.wait_send()
            f.wait_recv()
        cp_mine.wait()

    return pl.pallas_call(
        body,
        out_shape=jax.ShapeDtypeStruct((2 * m_per, n), jnp.bfloat16),
        in_specs=[pl.BlockSpec(memory_space=pl.ANY)],
        out_specs=pl.BlockSpec(memory_space=pl.ANY),
        scratch_shapes=[
            pltpu.VMEM((M_PER, N), jnp.float32),
            pltpu.VMEM((M_PER, N), jnp.bfloat16),
            pltpu.VMEM((M_PER, N), jnp.bfloat16),
            pltpu.SemaphoreType.DMA((KX,)),
            pltpu.SemaphoreType.DMA((KX,)),
            pltpu.SemaphoreType.DMA((KF,)),
            pltpu.SemaphoreType.DMA((KF,)),
            pltpu.SemaphoreType.DMA((KX + 1,)),
            pltpu.SemaphoreType.DMA((2,)),
            pltpu.SemaphoreType.DMA((KX,)),
        ],
        compiler_params=pltpu.CompilerParams(collective_id=0),
    )(pltpu.with_memory_space_constraint(x, pltpu.MemorySpace.HBM))
